# baseline (device time: 27274 ns/iter reference)
import jax
import jax.numpy as jnp
from jax import lax
from jax.experimental import pallas as pl
from jax.experimental.pallas import tpu as pltpu

N_DEV = 16
G = 4
NG = 4

_GELU_C = 0.7978845608028654


def _gelu(y):
    return 0.5 * y * (1.0 + jnp.tanh(_GELU_C * (y + 0.044715 * y * y * y)))


def kernel(x, w_mat):
    m_per, k = x.shape
    n = w_mat.shape[1]
    n_per = n // N_DEV
    n_grp = n // G

    def body(x_ref, w_hbm, out_ref, w_vmem, y_grp_ref, y_buf, comm_ref,
             copy_sems, send_sems, recv_sems):
        my = lax.axis_index("i")
        my_hi = my // NG
        my_lo = my % NG

        def w_copy(g, slot):
            gsel = (my_hi + g) % G
            return pltpu.make_async_copy(
                w_hbm.at[:, pl.ds(gsel * n_grp, n_grp)],
                w_vmem.at[slot],
                copy_sems.at[slot],
            )

        w_copy(0, 0).start()

        barrier = pltpu.get_barrier_semaphore()
        for d in range(N_DEV):
            @pl.when(my != d)
            def _():
                pl.semaphore_signal(
                    barrier, inc=1,
                    device_id=(d,), device_id_type=pl.DeviceIdType.MESH,
                )
        pl.semaphore_wait(barrier, N_DEV - 1)

        x_val = x_ref[:, :].astype(jnp.bfloat16)

        send_rdmas = []
        for g in range(G):
            slot = g % 2
            if g + 1 < G:
                w_copy(g + 1, (g + 1) % 2).start()
            w_copy(g, slot).wait()
            gsel = (my_hi + g) % G
            y_grp = _gelu(jnp.dot(x_val, w_vmem[slot].astype(jnp.bfloat16),
                                  preferred_element_type=jnp.float32))
            y_grp_ref[:, :] = y_grp.astype(jnp.bfloat16)
            for u in range(NG):
                t = NG * g + u
                jlo = (my_lo + u) % NG
                blk = y_grp_ref[:, pl.ds(jlo * n_per, n_per)]
                if t == 0:
                    out_ref[pl.ds(my * m_per, m_per), :] = blk.astype(jnp.float32)
                else:
                    j = NG * gsel + jlo
                    y_buf[t, :, :] = blk
                    rdma = pltpu.make_async_remote_copy(
                        src_ref=y_buf.at[t],
                        dst_ref=comm_ref.at[t],
                        send_sem=send_sems.at[t],
                        recv_sem=recv_sems.at[t],
                        device_id=(j,),
                        device_id_type=pl.DeviceIdType.MESH,
                    )
                    rdma.start()
                    send_rdmas.append(rdma)

        for t in range(1, N_DEV):
            g, u = t // NG, t % NG
            s = NG * ((my_hi - g) % G) + ((my_lo - u) % NG)
            recv = pltpu.make_async_remote_copy(
                src_ref=y_buf.at[t],
                dst_ref=comm_ref.at[t],
                send_sem=send_sems.at[t],
                recv_sem=recv_sems.at[t],
                device_id=(my,),
                device_id_type=pl.DeviceIdType.MESH,
            )
            recv.wait_recv()
            out_ref[pl.ds(s * m_per, m_per), :] = comm_ref[t].astype(jnp.float32)

        for rdma in send_rdmas:
            rdma.wait_send()

    out_shape = jax.ShapeDtypeStruct((N_DEV * m_per, n_per), jnp.float32)
    return pl.pallas_call(
        body,
        out_shape=out_shape,
        in_specs=[
            pl.BlockSpec(memory_space=pltpu.VMEM),
            pl.BlockSpec(memory_space=pltpu.MemorySpace.HBM),
        ],
        out_specs=pl.BlockSpec(memory_space=pltpu.VMEM),
        scratch_shapes=[
            pltpu.VMEM((2, k, n_grp), x.dtype),
            pltpu.VMEM((m_per, n_grp), jnp.bfloat16),
            pltpu.VMEM((N_DEV, m_per, n_per), jnp.bfloat16),
            pltpu.VMEM((N_DEV, m_per, n_per), jnp.bfloat16),
            pltpu.SemaphoreType.DMA((2,)),
            pltpu.SemaphoreType.DMA((N_DEV,)),
            pltpu.SemaphoreType.DMA((N_DEV,)),
        ],
        compiler_params=pltpu.CompilerParams(collective_id=0),
    )(x, w_mat)


# device time: 25967 ns/iter; 1.0503x vs baseline; 1.0503x over previous
import jax
import jax.numpy as jnp
from jax import lax
from jax.experimental import pallas as pl
from jax.experimental.pallas import tpu as pltpu

N_DEV = 16
G = 4
NG = 4

_GELU_C = 0.7978845608028654


def _gelu(y):
    return 0.5 * y * (1.0 + jnp.tanh(_GELU_C * (y + 0.044715 * y * y * y)))


def kernel(x, w_mat):
    m_per, k = x.shape
    n = w_mat.shape[1]
    n_per = n // N_DEV
    n_grp = n // G

    def body(x_ref, w_hbm, out_ref, w_vmem, y_grps, comm_ref,
             copy_sems, send_sems, recv_sems):
        my = lax.axis_index("i")
        my_hi = my // NG
        my_lo = my % NG

        def w_copy(g, slot):
            gsel = (my_hi + g) % G
            return pltpu.make_async_copy(
                w_hbm.at[:, pl.ds(gsel * n_grp, n_grp)],
                w_vmem.at[slot],
                copy_sems.at[slot],
            )

        w_copy(0, 0).start()

        barrier = pltpu.get_barrier_semaphore()
        for d in range(N_DEV):
            @pl.when(my != d)
            def _():
                pl.semaphore_signal(
                    barrier, inc=1,
                    device_id=(d,), device_id_type=pl.DeviceIdType.MESH,
                )
        pl.semaphore_wait(barrier, N_DEV - 1)

        x_val = x_ref[:, :].astype(jnp.bfloat16)

        send_rdmas = []

        def send_block(g, jlo):
            gsel = (my_hi + g) % G
            j = NG * gsel + jlo
            t = NG * g + (jlo - my_lo) % NG
            rdma = pltpu.make_async_remote_copy(
                src_ref=y_grps.at[g, :, pl.ds(jlo * n_per, n_per)],
                dst_ref=comm_ref.at[t],
                send_sem=send_sems.at[t],
                recv_sem=recv_sems.at[t],
                device_id=(j,),
                device_id_type=pl.DeviceIdType.MESH,
            )
            rdma.start()
            send_rdmas.append(rdma)

        pieces = [(0, 0, 2), (0, 2, 2),
                  (1, 0, 4), (2, 0, 4),
                  (3, 0, 2), (3, 2, 2)]
        started = set()
        for g, blo, nblk in pieces:
            slot = g % 2
            if g not in started:
                if g + 1 < G:
                    w_copy(g + 1, (g + 1) % 2).start()
                w_copy(g, slot).wait()
                started.add(g)
            cols = slice(blo * n_per, (blo + nblk) * n_per)
            y = _gelu(jnp.dot(x_val, w_vmem[slot][:, cols].astype(jnp.bfloat16),
                              preferred_element_type=jnp.float32))
            y_grps[g, :, cols] = y.astype(jnp.bfloat16)
            for jlo in range(blo, blo + nblk):
                send_block(g, jlo)

        for t in range(N_DEV):
            g, u = t // NG, t % NG
            s = NG * ((my_hi - g) % G) + ((my_lo - u) % NG)
            recv = pltpu.make_async_remote_copy(
                src_ref=comm_ref.at[t],
                dst_ref=comm_ref.at[t],
                send_sem=send_sems.at[t],
                recv_sem=recv_sems.at[t],
                device_id=(my,),
                device_id_type=pl.DeviceIdType.MESH,
            )
            recv.wait_recv()
            out_ref[pl.ds(s * m_per, m_per), :] = comm_ref[t].astype(jnp.float32)

        for rdma in send_rdmas:
            rdma.wait_send()

    out_shape = jax.ShapeDtypeStruct((N_DEV * m_per, n_per), jnp.float32)
    return pl.pallas_call(
        body,
        out_shape=out_shape,
        in_specs=[
            pl.BlockSpec(memory_space=pltpu.VMEM),
            pl.BlockSpec(memory_space=pltpu.MemorySpace.HBM),
        ],
        out_specs=pl.BlockSpec(memory_space=pltpu.VMEM),
        scratch_shapes=[
            pltpu.VMEM((2, k, n_grp), x.dtype),
            pltpu.VMEM((G, m_per, n_grp), jnp.bfloat16),
            pltpu.VMEM((N_DEV, m_per, n_per), jnp.bfloat16),
            pltpu.SemaphoreType.DMA((2,)),
            pltpu.SemaphoreType.DMA((N_DEV,)),
            pltpu.SemaphoreType.DMA((N_DEV,)),
        ],
        compiler_params=pltpu.CompilerParams(collective_id=0),
    )(x, w_mat)
